# Initial kernel scaffold; baseline (speedup 1.0000x reference)
import functools

import jax
import jax.numpy as jnp
from jax import lax
from jax.experimental import pallas as pl
from jax.experimental.pallas import tpu as pltpu
from jax.experimental.pallas import tpu_sc as plsc

NC = 2
NS = 16
NW = NC * NS

LANES = 16


def _sc_gather_mean(emb, nodes3, neighs3, *, BP, BPW, D, DEG, SCH, NG):
    GR = 128 // DEG

    def body(emb_hbm, nodes_hbm, neighs_hbm, self_hbm, mean_hbm,
             nidx, sidx, nbuf, sbuf, obuf, sem):
        wid = lax.axis_index("s") * NC + lax.axis_index("c")
        base = wid * BPW
        pltpu.sync_copy(neighs_hbm.at[wid], nidx)
        pltpu.sync_copy(nodes_hbm.at[wid], sidx)

        def self_chunk(c, carry):
            pltpu.async_copy(emb_hbm.at[sidx.at[c]], sbuf, sem).wait()
            pltpu.sync_copy(sbuf, self_hbm.at[pl.ds(base + c * 64, 64)])
            return carry
        lax.fori_loop(0, SCH, self_chunk, 0)

        inv = jnp.float32(1.0 / DEG)

        def grp(g, carry):
            pltpu.async_copy(emb_hbm.at[nidx.at[g]], nbuf, sem).wait()
            for r in range(GR):
                row = g * GR + r
                for cg in range(D // LANES):
                    sl = pl.ds(cg * LANES, LANES)
                    acc = nbuf[r * DEG, sl]
                    for j in range(1, DEG):
                        acc = acc + nbuf[r * DEG + j, sl]
                    obuf[row, sl] = acc * inv
            return carry
        lax.fori_loop(0, NG, grp, 0)

        pltpu.sync_copy(obuf, mean_hbm.at[pl.ds(base, BPW)])

    mesh = plsc.VectorSubcoreMesh(core_axis_name="c", subcore_axis_name="s")
    fn = pl.kernel(
        body,
        out_type=[
            jax.ShapeDtypeStruct((BP, D), jnp.float32),
            jax.ShapeDtypeStruct((BP, D), jnp.float32),
        ],
        mesh=mesh,
        scratch_types=[
            pltpu.VMEM((NG, 128), jnp.int32),
            pltpu.VMEM((SCH, 64), jnp.int32),
            pltpu.VMEM((128, D), jnp.float32),
            pltpu.VMEM((64, D), jnp.float32),
            pltpu.VMEM((BPW, D), jnp.float32),
            pltpu.SemaphoreType.DMA,
        ],
    )
    return fn(emb, nodes3, neighs3)


def _tc_combine(self_raw, neigh_mean, WaT, WbT, b2, *, BP, D, RB):
    def body(s_ref, n_ref, wa_ref, wb_ref, b_ref, o_ref):
        s = s_ref[...].astype(jnp.float16).astype(jnp.float32)
        acc = jnp.dot(s, wa_ref[...], preferred_element_type=jnp.float32)
        acc = acc + jnp.dot(n_ref[...], wb_ref[...],
                            preferred_element_type=jnp.float32)
        o_ref[...] = jnp.maximum(acc + b_ref[...], 0.0)

    return pl.pallas_call(
        body,
        grid=(BP // RB,),
        in_specs=[
            pl.BlockSpec((RB, D), lambda i: (i, 0)),
            pl.BlockSpec((RB, D), lambda i: (i, 0)),
            pl.BlockSpec((D, D), lambda i: (0, 0)),
            pl.BlockSpec((D, D), lambda i: (0, 0)),
            pl.BlockSpec((1, D), lambda i: (0, 0)),
        ],
        out_specs=pl.BlockSpec((RB, D), lambda i: (i, 0)),
        out_shape=jax.ShapeDtypeStruct((BP, D), jnp.float32),
    )(self_raw, neigh_mean, WaT, WbT, b2)


def kernel(nodes, neighs, emb, W1, b1):
    B = nodes.shape[0]
    DEG = neighs.shape[1]
    D = emb.shape[1]

    BPW = -(-B // NW)
    BPW = -(-BPW // 64) * 64
    BP = NW * BPW
    SCH = BPW // 64
    NG = BPW * DEG // 128

    pad = BP - B
    nodes_p = jnp.pad(nodes, (0, pad))
    neighs_p = jnp.pad(neighs, ((0, pad), (0, 0)))
    nodes3 = nodes_p.reshape(NW, SCH, 64)
    neighs3 = neighs_p.reshape(NW, NG, 128)

    self_raw, neigh_mean = _sc_gather_mean(
        emb, nodes3, neighs3, BP=BP, BPW=BPW, D=D, DEG=DEG, SCH=SCH, NG=NG)

    WaT = W1[:, :D].T
    WbT = W1[:, D:].T
    b2 = b1.reshape(1, D)
    out = _tc_combine(self_raw, neigh_mean, WaT, WbT, b2, BP=BP, D=D, RB=512)
    return out[:B]


# speedup vs baseline: 1.2148x; 1.2148x over previous
import functools

import jax
import jax.numpy as jnp
from jax import lax
from jax.experimental import pallas as pl
from jax.experimental.pallas import tpu as pltpu
from jax.experimental.pallas import tpu_sc as plsc

NC = 2
NS = 16
NW = NC * NS

LANES = 16


def _sc_gather_mean(emb, nodes3, neighs3, *, BP, BPW, D, DEG, SCH, NG):
    GR = 128 // DEG

    def body(emb_hbm, nodes_hbm, neighs_hbm, self_hbm, mean_hbm,
             nidx, sidx, nbuf, sbuf, obuf, sem):
        wid = lax.axis_index("s") * NC + lax.axis_index("c")
        base = wid * BPW
        pltpu.sync_copy(neighs_hbm.at[wid], nidx)
        pltpu.sync_copy(nodes_hbm.at[wid], sidx)

        def self_chunk(c, carry):
            pltpu.async_copy(emb_hbm.at[sidx.at[c]], sbuf, sem).wait()
            pltpu.sync_copy(sbuf, self_hbm.at[pl.ds(base + c * 64, 64)])
            return carry
        lax.fori_loop(0, SCH, self_chunk, 0)

        inv = jnp.float32(1.0 / DEG)

        def grp(g, carry):
            pltpu.async_copy(emb_hbm.at[nidx.at[g]], nbuf, sem).wait()
            for r in range(GR):
                row = g * GR + r
                for cg in range(D // LANES):
                    sl = pl.ds(cg * LANES, LANES)
                    acc = nbuf[r * DEG, sl]
                    for j in range(1, DEG):
                        acc = acc + nbuf[r * DEG + j, sl]
                    obuf[row, sl] = acc * inv
            return carry
        lax.fori_loop(0, NG, grp, 0)

        pltpu.sync_copy(obuf, mean_hbm.at[pl.ds(base, BPW)])

    mesh = plsc.VectorSubcoreMesh(core_axis_name="c", subcore_axis_name="s")
    fn = pl.kernel(
        body,
        out_type=[
            jax.ShapeDtypeStruct((BP, D), jnp.float32),
            jax.ShapeDtypeStruct((BP, D), jnp.float32),
        ],
        mesh=mesh,
        scratch_types=[
            pltpu.VMEM((NG, 128), jnp.int32),
            pltpu.VMEM((SCH, 64), jnp.int32),
            pltpu.VMEM((128, D), jnp.float32),
            pltpu.VMEM((64, D), jnp.float32),
            pltpu.VMEM((BPW, D), jnp.float32),
            pltpu.SemaphoreType.DMA,
        ],
    )
    return fn(emb, nodes3, neighs3)


def _tc_combine(self_raw, neigh_mean, WaT, WbT, b2, *, BP, D, RB):
    def body(s_ref, n_ref, wa_ref, wb_ref, b_ref, o_ref):
        acc = jnp.dot(s_ref[...], wa_ref[...],
                      preferred_element_type=jnp.float32)
        acc = acc + jnp.dot(n_ref[...], wb_ref[...],
                            preferred_element_type=jnp.float32)
        o_ref[...] = jnp.maximum(acc + b_ref[...], 0.0)

    return pl.pallas_call(
        body,
        grid=(BP // RB,),
        in_specs=[
            pl.BlockSpec((RB, D), lambda i: (i, 0)),
            pl.BlockSpec((RB, D), lambda i: (i, 0)),
            pl.BlockSpec((D, D), lambda i: (0, 0)),
            pl.BlockSpec((D, D), lambda i: (0, 0)),
            pl.BlockSpec((1, D), lambda i: (0, 0)),
        ],
        out_specs=pl.BlockSpec((RB, D), lambda i: (i, 0)),
        out_shape=jax.ShapeDtypeStruct((BP, D), jnp.float32),
    )(self_raw, neigh_mean, WaT, WbT, b2)


def kernel(nodes, neighs, emb, W1, b1):
    B = nodes.shape[0]
    DEG = neighs.shape[1]
    D = emb.shape[1]

    BPW = -(-B // NW)
    BPW = -(-BPW // 64) * 64
    BP = NW * BPW
    SCH = BPW // 64
    NG = BPW * DEG // 128

    pad = BP - B
    nodes_p = jnp.pad(nodes, (0, pad))
    neighs_p = jnp.pad(neighs, ((0, pad), (0, 0)))
    nodes3 = nodes_p.reshape(NW, SCH, 64)
    neighs3 = neighs_p.reshape(NW, NG, 128)

    self_raw, neigh_mean = _sc_gather_mean(
        emb, nodes3, neighs3, BP=BP, BPW=BPW, D=D, DEG=DEG, SCH=SCH, NG=NG)
    self_raw = self_raw.astype(jnp.float16).astype(jnp.float32)

    WaT = W1[:, :D].T
    WbT = W1[:, D:].T
    b2 = b1.reshape(1, D)
    out = _tc_combine(self_raw, neigh_mean, WaT, WbT, b2, BP=BP, D=D, RB=512)
    return out[:B]


# speedup vs baseline: 1.6054x; 1.3215x over previous
import functools

import jax
import jax.numpy as jnp
from jax import lax
from jax.experimental import pallas as pl
from jax.experimental.pallas import tpu as pltpu
from jax.experimental.pallas import tpu_sc as plsc

NC = 2
NS = 16
NW = NC * NS

LANES = 16


def _sc_gather_mean(emb, nodes3, neighs3, *, BP, BPW, D, DEG, SCH, NG):
    GR = 128 // DEG

    NBUF = 2

    def body(emb_hbm, nodes_hbm, neighs_hbm, self_hbm, mean_hbm,
             nidx, sidx, nbuf, selfbuf, obuf, nsem0, nsem1, ssem):
        nsems = (nsem0, nsem1)
        wid = lax.axis_index("s") * NC + lax.axis_index("c")
        base = wid * BPW
        pltpu.sync_copy(neighs_hbm.at[wid], nidx)
        pltpu.sync_copy(nodes_hbm.at[wid], sidx)

        for c in range(SCH):
            pltpu.async_copy(emb_hbm.at[sidx.at[c]],
                             selfbuf.at[pl.ds(c * 64, 64)], ssem)

        for b in range(NBUF):
            pltpu.async_copy(emb_hbm.at[nidx.at[b]], nbuf.at[b], nsems[b])

        inv = jnp.float32(1.0 / DEG)

        def grp(i, carry):
            for b in range(NBUF):
                g = i * NBUF + b
                pltpu.make_async_copy(emb_hbm.at[nidx.at[g]],
                                      nbuf.at[b], nsems[b]).wait()
                for r in range(GR):
                    row = g * GR + r
                    for cg in range(D // LANES):
                        sl = pl.ds(cg * LANES, LANES)
                        acc = nbuf[b, r * DEG, sl]
                        for j in range(1, DEG):
                            acc = acc + nbuf[b, r * DEG + j, sl]
                        obuf[row, sl] = acc * inv

                @pl.when(g + NBUF < NG)
                def _():
                    pltpu.async_copy(emb_hbm.at[nidx.at[g + NBUF]],
                                     nbuf.at[b], nsems[b])
            return carry
        lax.fori_loop(0, NG // NBUF, grp, 0)

        for c in range(SCH):
            pltpu.make_async_copy(emb_hbm.at[sidx.at[c]],
                                  selfbuf.at[pl.ds(c * 64, 64)], ssem).wait()
        pltpu.sync_copy(selfbuf, self_hbm.at[pl.ds(base, BPW)])
        pltpu.sync_copy(obuf, mean_hbm.at[pl.ds(base, BPW)])

    mesh = plsc.VectorSubcoreMesh(core_axis_name="c", subcore_axis_name="s")
    fn = pl.kernel(
        body,
        out_type=[
            jax.ShapeDtypeStruct((BP, D), jnp.float32),
            jax.ShapeDtypeStruct((BP, D), jnp.float32),
        ],
        mesh=mesh,
        scratch_types=[
            pltpu.VMEM((NG, 128), jnp.int32),
            pltpu.VMEM((SCH, 64), jnp.int32),
            pltpu.VMEM((NBUF, 128, D), jnp.float32),
            pltpu.VMEM((BPW, D), jnp.float32),
            pltpu.VMEM((BPW, D), jnp.float32),
            pltpu.SemaphoreType.DMA,
            pltpu.SemaphoreType.DMA,
            pltpu.SemaphoreType.DMA,
        ],
    )
    return fn(emb, nodes3, neighs3)


def _tc_combine(self_raw, neigh_mean, WaT, WbT, b2, *, BP, D, RB):
    def body(s_ref, n_ref, wa_ref, wb_ref, b_ref, o_ref):
        acc = jnp.dot(s_ref[...], wa_ref[...],
                      preferred_element_type=jnp.float32)
        acc = acc + jnp.dot(n_ref[...], wb_ref[...],
                            preferred_element_type=jnp.float32)
        o_ref[...] = jnp.maximum(acc + b_ref[...], 0.0)

    return pl.pallas_call(
        body,
        grid=(BP // RB,),
        in_specs=[
            pl.BlockSpec((RB, D), lambda i: (i, 0)),
            pl.BlockSpec((RB, D), lambda i: (i, 0)),
            pl.BlockSpec((D, D), lambda i: (0, 0)),
            pl.BlockSpec((D, D), lambda i: (0, 0)),
            pl.BlockSpec((1, D), lambda i: (0, 0)),
        ],
        out_specs=pl.BlockSpec((RB, D), lambda i: (i, 0)),
        out_shape=jax.ShapeDtypeStruct((BP, D), jnp.float32),
    )(self_raw, neigh_mean, WaT, WbT, b2)


def kernel(nodes, neighs, emb, W1, b1):
    B = nodes.shape[0]
    DEG = neighs.shape[1]
    D = emb.shape[1]

    BPW = -(-B // NW)
    BPW = -(-BPW // 64) * 64
    BP = NW * BPW
    SCH = BPW // 64
    NG = BPW * DEG // 128

    pad = BP - B
    nodes_p = jnp.pad(nodes, (0, pad))
    neighs_p = jnp.pad(neighs, ((0, pad), (0, 0)))
    nodes3 = nodes_p.reshape(NW, SCH, 64)
    neighs3 = neighs_p.reshape(NW, NG, 128)

    self_raw, neigh_mean = _sc_gather_mean(
        emb, nodes3, neighs3, BP=BP, BPW=BPW, D=D, DEG=DEG, SCH=SCH, NG=NG)
    self_raw = self_raw.astype(jnp.float16).astype(jnp.float32)

    WaT = W1[:, :D].T
    WbT = W1[:, D:].T
    b2 = b1.reshape(1, D)
    out = _tc_combine(self_raw, neigh_mean, WaT, WbT, b2, BP=BP, D=D, RB=512)
    return out[:B]


# speedup vs baseline: 1.6192x; 1.0086x over previous
import functools

import jax
import jax.numpy as jnp
from jax import lax
from jax.experimental import pallas as pl
from jax.experimental.pallas import tpu as pltpu
from jax.experimental.pallas import tpu_sc as plsc

NC = 2
NS = 16
NW = NC * NS

LANES = 16


def _sc_gather_mean(emb, nodes3, neighs3, *, BP, BPW, D, DEG, SCH, NG):
    GI = 64
    GR = GI // DEG

    NBUF = 5
    assert NG % NBUF == 0

    def body(emb_hbm, nodes_hbm, neighs_hbm, self_hbm, mean_hbm,
             nidx, sidx, nbuf, selfbuf, ostage, *sems):
        nsems = sems[:NBUF]
        ssem, osem = sems[NBUF], sems[NBUF + 1]
        wid = lax.axis_index("s") * NC + lax.axis_index("c")
        base = wid * BPW
        pltpu.sync_copy(neighs_hbm.at[wid], nidx)
        pltpu.sync_copy(nodes_hbm.at[wid], sidx)

        for c in range(SCH):
            pltpu.async_copy(emb_hbm.at[sidx.at[c]],
                             selfbuf.at[pl.ds(c * 64, 64)], ssem)

        for b in range(NBUF):
            pltpu.async_copy(emb_hbm.at[nidx.at[b]], nbuf.at[b], nsems[b])

        inv = jnp.float32(1.0 / DEG)

        def out_copy(g, b):
            return pltpu.make_async_copy(
                ostage.at[b], mean_hbm.at[pl.ds(base + g * GR, GR)], osem)

        def grp(i, carry):
            for b in range(NBUF):
                g = i * NBUF + b
                pltpu.make_async_copy(emb_hbm.at[nidx.at[g]],
                                      nbuf.at[b], nsems[b]).wait()

                @pl.when(g >= NBUF)
                def _():
                    out_copy(g - NBUF, b).wait()

                for r in range(GR):
                    for cg in range(D // LANES):
                        sl = pl.ds(cg * LANES, LANES)
                        acc = nbuf[b, r * DEG, sl]
                        for j in range(1, DEG):
                            acc = acc + nbuf[b, r * DEG + j, sl]
                        ostage[b, r, sl] = acc * inv
                out_copy(g, b).start()

                @pl.when(g + NBUF < NG)
                def _():
                    pltpu.async_copy(emb_hbm.at[nidx.at[g + NBUF]],
                                     nbuf.at[b], nsems[b])
            return carry
        lax.fori_loop(0, NG // NBUF, grp, 0)

        for b in range(NBUF):
            out_copy(NG - NBUF + b, b).wait()
        for c in range(SCH):
            pltpu.make_async_copy(emb_hbm.at[sidx.at[c]],
                                  selfbuf.at[pl.ds(c * 64, 64)], ssem).wait()
        pltpu.sync_copy(selfbuf, self_hbm.at[pl.ds(base, BPW)])

    mesh = plsc.VectorSubcoreMesh(core_axis_name="c", subcore_axis_name="s")
    fn = pl.kernel(
        body,
        out_type=[
            jax.ShapeDtypeStruct((BP, D), jnp.float32),
            jax.ShapeDtypeStruct((BP, D), jnp.float32),
        ],
        mesh=mesh,
        scratch_types=[
            pltpu.VMEM((NG, GI), jnp.int32),
            pltpu.VMEM((SCH, 64), jnp.int32),
            pltpu.VMEM((NBUF, GI, D), jnp.float32),
            pltpu.VMEM((BPW, D), jnp.float32),
            pltpu.VMEM((NBUF, GR, D), jnp.float32),
        ] + [pltpu.SemaphoreType.DMA] * (NBUF + 2),
    )
    return fn(emb, nodes3, neighs3)


def _tc_combine(self_raw, neigh_mean, WaT, WbT, b2, *, BP, D, RB):
    def body(s_ref, n_ref, wa_ref, wb_ref, b_ref, o_ref):
        acc = jnp.dot(s_ref[...], wa_ref[...],
                      preferred_element_type=jnp.float32)
        acc = acc + jnp.dot(n_ref[...], wb_ref[...],
                            preferred_element_type=jnp.float32)
        o_ref[...] = jnp.maximum(acc + b_ref[...], 0.0)

    return pl.pallas_call(
        body,
        grid=(BP // RB,),
        in_specs=[
            pl.BlockSpec((RB, D), lambda i: (i, 0)),
            pl.BlockSpec((RB, D), lambda i: (i, 0)),
            pl.BlockSpec((D, D), lambda i: (0, 0)),
            pl.BlockSpec((D, D), lambda i: (0, 0)),
            pl.BlockSpec((1, D), lambda i: (0, 0)),
        ],
        out_specs=pl.BlockSpec((RB, D), lambda i: (i, 0)),
        out_shape=jax.ShapeDtypeStruct((BP, D), jnp.float32),
    )(self_raw, neigh_mean, WaT, WbT, b2)


def kernel(nodes, neighs, emb, W1, b1):
    B = nodes.shape[0]
    DEG = neighs.shape[1]
    D = emb.shape[1]

    BPW = -(-B // NW)
    BPW = -(-BPW // 64) * 64
    BP = NW * BPW
    SCH = BPW // 64
    NG = BPW * DEG // 64

    pad = BP - B
    nodes_p = jnp.pad(nodes, (0, pad))
    neighs_p = jnp.pad(neighs, ((0, pad), (0, 0)))
    nodes3 = nodes_p.reshape(NW, SCH, 64)
    neighs3 = neighs_p.reshape(NW, NG, 64)

    self_raw, neigh_mean = _sc_gather_mean(
        emb, nodes3, neighs3, BP=BP, BPW=BPW, D=D, DEG=DEG, SCH=SCH, NG=NG)
    self_raw = self_raw.astype(jnp.float16).astype(jnp.float32)

    WaT = W1[:, :D].T
    WbT = W1[:, D:].T
    b2 = b1.reshape(1, D)
    out = _tc_combine(self_raw, neigh_mean, WaT, WbT, b2, BP=BP, D=D, RB=512)
    return out[:B]


# speedup vs baseline: 5.5322x; 3.4165x over previous
import functools

import numpy as np

import jax
import jax.numpy as jnp
from jax import lax
from jax.experimental import pallas as pl
from jax.experimental.pallas import tpu as pltpu
from jax.experimental.pallas import tpu_sc as plsc

NC = 2
NS = 16
NW = NC * NS

LANES = 16


def _sc_gather_mean(embp, nodes3, neighs3, *, BP, BPW, D, DEG, SCH, NG):
    GI = DEG
    SC_ROWS = 16
    NCG = D // LANES
    NQ = 4
    QG = NG // NQ

    NBUF = 2
    assert QG % NBUF == 0 and SCH >= 2 and SCH % 2 == 0

    V = embp.shape[0]
    VCH = -(-V // NS)
    VCH = -(-VCH // 8) * 8
    VLAST = V - (NS - 1) * VCH
    assert VLAST > 0 and VLAST % 8 == 0

    def body(emb_hbm, nodes_hbm, neighs_hbm, self_hbm, mean_hbm,
             table, nidx, sidx, nbuf, sring, ostage, *sems):
        nsems = sems[:NBUF]
        ssem, osem, isem = sems[NBUF], sems[NBUF + 1], sems[NBUF + 2]
        sid = lax.axis_index("s")
        wid = sid * NC + lax.axis_index("c")
        base = wid * BPW

        vlo = sid * VCH

        @pl.when(sid < NS - 1)
        def _():
            pltpu.sync_copy(emb_hbm.at[pl.ds(vlo, VCH)],
                            table.at[pl.ds(vlo, VCH)])

        @pl.when(sid == NS - 1)
        def _():
            pltpu.sync_copy(emb_hbm.at[pl.ds((NS - 1) * VCH, VLAST)],
                            table.at[pl.ds((NS - 1) * VCH, VLAST)])

        pltpu.sync_copy(neighs_hbm.at[wid, 0], nidx.at[0])
        pltpu.sync_copy(nodes_hbm.at[wid], sidx)

        for c in range(2):
            pltpu.async_copy(emb_hbm.at[sidx.at[c]], sring.at[c % 2], ssem)

        plsc.subcore_barrier()

        inv = jnp.float32(1.0 / DEG)

        for q in range(NQ):
            qidx = nidx.at[q % 2]
            qbase = base + q * QG
            if q > 0:
                pltpu.make_async_copy(neighs_hbm.at[wid, q],
                                      nidx.at[q % 2], isem).wait()
            if q + 1 < NQ:
                pltpu.async_copy(neighs_hbm.at[wid, q + 1],
                                 nidx.at[(q + 1) % 2], isem)

            for b in range(NBUF):
                pltpu.async_copy(table.at[qidx.at[b]], nbuf.at[b], nsems[b])

            def out_copy(g, b):
                return pltpu.make_async_copy(
                    ostage.at[b], mean_hbm.at[pl.ds(qbase + g, 1)], osem)

            def grp(i, carry):
                for b in range(NBUF):
                    g = i * NBUF + b
                    pltpu.make_async_copy(table.at[qidx.at[g]],
                                          nbuf.at[b], nsems[b]).wait()

                    @pl.when(g >= NBUF)
                    def _():
                        out_copy(g - NBUF, b).wait()

                    def red(j, accs):
                        return tuple(
                            accs[p] + nbuf[b, j, pl.ds(p * LANES, LANES)]
                            for p in range(NCG))
                    accs = tuple(
                        nbuf[b, 0, pl.ds(p * LANES, LANES)]
                        for p in range(NCG))
                    accs = lax.fori_loop(1, DEG, red, accs, unroll=4)
                    for p in range(NCG):
                        ostage[b, 0, pl.ds(p * LANES, LANES)] = accs[p] * inv
                    out_copy(g, b).start()

                    @pl.when(g + NBUF < QG)
                    def _():
                        pltpu.async_copy(table.at[qidx.at[g + NBUF]],
                                         nbuf.at[b], nsems[b])
                return carry
            lax.fori_loop(0, QG // NBUF, grp, 0)

            for b in range(NBUF):
                out_copy(QG - NBUF + b, b).wait()

        def self_tail(i, carry):
            for b in range(2):
                c = i * 2 + b
                pltpu.make_async_copy(emb_hbm.at[sidx.at[c]],
                                      sring.at[b], ssem).wait()
                pltpu.sync_copy(
                    sring.at[b],
                    self_hbm.at[pl.ds(base + c * SC_ROWS, SC_ROWS)])

                @pl.when(c + 2 < SCH)
                def _():
                    pltpu.async_copy(emb_hbm.at[sidx.at[c + 2]],
                                     sring.at[b], ssem)
            return carry
        lax.fori_loop(0, SCH // 2, self_tail, 0)

    mesh = plsc.VectorSubcoreMesh(core_axis_name="c", subcore_axis_name="s")
    fn = pl.kernel(
        body,
        out_type=[
            jax.ShapeDtypeStruct((BP, D), jnp.float32),
            jax.ShapeDtypeStruct((BP, D), jnp.float32),
        ],
        mesh=mesh,
        scratch_types=[
            pltpu.VMEM_SHARED((V, D), jnp.float32),
            pltpu.VMEM((2, QG, GI), jnp.int32),
            pltpu.VMEM((SCH, SC_ROWS), jnp.int32),
            pltpu.VMEM((NBUF, GI, D), jnp.float32),
            pltpu.VMEM((2, SC_ROWS, D), jnp.float32),
            pltpu.VMEM((NBUF, 1, D), jnp.float32),
        ] + [pltpu.SemaphoreType.DMA] * (NBUF + 3),
    )
    return fn(embp, nodes3, neighs3)


def _tc_combine(self_raw, neigh_mean, WaT, WbT, b2, *, BP, D, RB):
    def body(s_ref, n_ref, wa_ref, wb_ref, b_ref, o_ref):
        acc = jnp.dot(s_ref[...], wa_ref[...],
                      preferred_element_type=jnp.float32)
        acc = acc + jnp.dot(n_ref[...], wb_ref[...],
                            preferred_element_type=jnp.float32)
        o_ref[...] = jnp.maximum(acc + b_ref[...], 0.0)

    return pl.pallas_call(
        body,
        grid=(BP // RB,),
        in_specs=[
            pl.BlockSpec((RB, D), lambda i: (i, 0)),
            pl.BlockSpec((RB, D), lambda i: (i, 0)),
            pl.BlockSpec((D, D), lambda i: (0, 0)),
            pl.BlockSpec((D, D), lambda i: (0, 0)),
            pl.BlockSpec((1, D), lambda i: (0, 0)),
        ],
        out_specs=pl.BlockSpec((RB, D), lambda i: (i, 0)),
        out_shape=jax.ShapeDtypeStruct((BP, D), jnp.float32),
    )(self_raw, neigh_mean, WaT, WbT, b2)


def kernel(nodes, neighs, emb, W1, b1):
    B = nodes.shape[0]
    DEG = neighs.shape[1]
    D = emb.shape[1]

    BPW = -(-B // NW)
    BPW = -(-BPW // 64) * 64
    BP = NW * BPW
    SCH = BPW // 16
    NG = BPW

    embp = emb
    if embp.shape[0] % 8:
        embp = jnp.pad(embp, ((0, 8 - embp.shape[0] % 8), (0, 0)))

    pad = BP - B
    nodes_p = jnp.pad(nodes, (0, pad))
    neighs_p = jnp.pad(neighs, ((0, pad), (0, 0)))
    nodes3 = nodes_p.reshape(NW, SCH, 16)
    neighs3 = neighs_p.reshape(NW, 4, NG // 4, DEG)

    self_raw, neigh_mean = _sc_gather_mean(
        embp, nodes3, neighs3, BP=BP, BPW=BPW, D=D, DEG=DEG, SCH=SCH, NG=NG)
    self_raw = self_raw.astype(jnp.float16).astype(jnp.float32)

    WaT = W1[:, :D].T
    WbT = W1[:, D:].T
    b2 = b1.reshape(1, D)
    out = _tc_combine(self_raw, neigh_mean, WaT, WbT, b2, BP=BP, D=D, RB=512)
    return out[:B]


# speedup vs baseline: 5.7847x; 1.0457x over previous
import functools

import numpy as np

import jax
import jax.numpy as jnp
from jax import lax
from jax.experimental import pallas as pl
from jax.experimental.pallas import tpu as pltpu
from jax.experimental.pallas import tpu_sc as plsc

NC = 2
NS = 16
NW = NC * NS

LANES = 16


def _sc_gather_mean(embp, nodes3, neighs3, *, BP, BPW, D, DEG, SCH, NG):
    GI = DEG
    SC_ROWS = 16
    NCG = D // LANES
    NQ = 4
    QG = NG // NQ

    NBUF = 2
    assert QG % NBUF == 0 and SCH >= 2 and SCH % 2 == 0

    V = embp.shape[0]
    VCH = -(-V // NS)
    VCH = -(-VCH // 8) * 8
    VLAST = V - (NS - 1) * VCH
    assert VLAST > 0 and VLAST % 8 == 0

    def body(emb_hbm, nodes_hbm, neighs_hbm, self_hbm, mean_hbm,
             table, nidx, sidx, nbuf, sring, ostage, *sems):
        nsems = sems[:NBUF]
        ssem, osem, isem = sems[NBUF], sems[NBUF + 1], sems[NBUF + 2]
        sid = lax.axis_index("s")
        wid = sid * NC + lax.axis_index("c")
        base = wid * BPW

        vlo = sid * VCH

        @pl.when(sid < NS - 1)
        def _():
            pltpu.sync_copy(emb_hbm.at[pl.ds(vlo, VCH)],
                            table.at[pl.ds(vlo, VCH)])

        @pl.when(sid == NS - 1)
        def _():
            pltpu.sync_copy(emb_hbm.at[pl.ds((NS - 1) * VCH, VLAST)],
                            table.at[pl.ds((NS - 1) * VCH, VLAST)])

        pltpu.sync_copy(neighs_hbm.at[wid, 0], nidx.at[0])
        pltpu.sync_copy(nodes_hbm.at[wid], sidx)

        for c in range(2):
            pltpu.async_copy(emb_hbm.at[sidx.at[c]], sring.at[c % 2], ssem)

        plsc.subcore_barrier()

        inv = jnp.float32(1.0 / DEG)

        for q in range(NQ):
            qidx = nidx.at[q % 2]
            qbase = base + q * QG
            if q > 0:
                pltpu.make_async_copy(neighs_hbm.at[wid, q],
                                      nidx.at[q % 2], isem).wait()
            if q + 1 < NQ:
                pltpu.async_copy(neighs_hbm.at[wid, q + 1],
                                 nidx.at[(q + 1) % 2], isem)

            for b in range(NBUF):
                pltpu.async_copy(table.at[qidx.at[b]], nbuf.at[b], nsems[b])

            def out_copy(g, b):
                return pltpu.make_async_copy(
                    ostage.at[b], mean_hbm.at[pl.ds(qbase + g, 1)], osem)

            def grp(i, carry):
                for b in range(NBUF):
                    g = i * NBUF + b
                    pltpu.make_async_copy(table.at[qidx.at[g]],
                                          nbuf.at[b], nsems[b]).wait()

                    @pl.when(g >= NBUF)
                    def _():
                        out_copy(g - NBUF, b).wait()

                    def red(j, accs):
                        return tuple(
                            accs[p] + nbuf[b, j, pl.ds(p * LANES, LANES)]
                            for p in range(NCG))
                    accs = tuple(
                        nbuf[b, 0, pl.ds(p * LANES, LANES)]
                        for p in range(NCG))
                    accs = lax.fori_loop(1, DEG, red, accs, unroll=4)
                    for p in range(NCG):
                        ostage[b, 0, pl.ds(p * LANES, LANES)] = accs[p] * inv
                    out_copy(g, b).start()

                    @pl.when(g + NBUF < QG)
                    def _():
                        pltpu.async_copy(table.at[qidx.at[g + NBUF]],
                                         nbuf.at[b], nsems[b])
                return carry
            lax.fori_loop(0, QG // NBUF, grp, 0)

            for b in range(NBUF):
                out_copy(QG - NBUF + b, b).wait()

        def self_tail(i, carry):
            for b in range(2):
                c = i * 2 + b
                pltpu.make_async_copy(emb_hbm.at[sidx.at[c]],
                                      sring.at[b], ssem).wait()
                pltpu.sync_copy(
                    sring.at[b],
                    self_hbm.at[pl.ds(base + c * SC_ROWS, SC_ROWS)])

                @pl.when(c + 2 < SCH)
                def _():
                    pltpu.async_copy(emb_hbm.at[sidx.at[c + 2]],
                                     sring.at[b], ssem)
            return carry
        lax.fori_loop(0, SCH // 2, self_tail, 0)

    mesh = plsc.VectorSubcoreMesh(core_axis_name="c", subcore_axis_name="s")
    fn = pl.kernel(
        body,
        out_type=[
            jax.ShapeDtypeStruct((BP, D), jnp.float32),
            jax.ShapeDtypeStruct((BP, D), jnp.float32),
        ],
        mesh=mesh,
        scratch_types=[
            pltpu.VMEM_SHARED((V, D), jnp.float32),
            pltpu.VMEM((2, QG, GI), jnp.int32),
            pltpu.VMEM((SCH, SC_ROWS), jnp.int32),
            pltpu.VMEM((NBUF, GI, D), jnp.float32),
            pltpu.VMEM((2, SC_ROWS, D), jnp.float32),
            pltpu.VMEM((NBUF, 1, D), jnp.float32),
        ] + [pltpu.SemaphoreType.DMA] * (NBUF + 3),
    )
    return fn(embp, nodes3, neighs3)


def _tc_combine(self_raw, neigh_mean, WaT, WbT, b2, *, B, D, RB):
    def body(s_ref, n_ref, wa_ref, wb_ref, b_ref, o_ref):
        w = lax.bitcast_convert_type(s_ref[...], jnp.int32)
        w = w + 0x0FFF + ((w >> 13) & 1)
        s = lax.bitcast_convert_type(w & ~0x1FFF, jnp.float32)
        acc = jnp.dot(s, wa_ref[...], preferred_element_type=jnp.float32)
        acc = acc + jnp.dot(n_ref[...], wb_ref[...],
                            preferred_element_type=jnp.float32)
        o_ref[...] = jnp.maximum(acc + b_ref[...], 0.0)

    return pl.pallas_call(
        body,
        grid=(B // RB,),
        in_specs=[
            pl.BlockSpec((RB, D), lambda i: (i, 0)),
            pl.BlockSpec((RB, D), lambda i: (i, 0)),
            pl.BlockSpec((D, D), lambda i: (0, 0)),
            pl.BlockSpec((D, D), lambda i: (0, 0)),
            pl.BlockSpec((1, D), lambda i: (0, 0)),
        ],
        out_specs=pl.BlockSpec((RB, D), lambda i: (i, 0)),
        out_shape=jax.ShapeDtypeStruct((B, D), jnp.float32),
    )(self_raw, neigh_mean, WaT, WbT, b2)


def kernel(nodes, neighs, emb, W1, b1):
    B = nodes.shape[0]
    DEG = neighs.shape[1]
    D = emb.shape[1]

    BPW = -(-B // NW)
    BPW = -(-BPW // 64) * 64
    BP = NW * BPW
    SCH = BPW // 16
    NG = BPW

    embp = emb
    if embp.shape[0] % 8:
        embp = jnp.pad(embp, ((0, 8 - embp.shape[0] % 8), (0, 0)))

    pad = BP - B
    nodes_p = jnp.pad(nodes, (0, pad))
    neighs_p = jnp.pad(neighs, ((0, pad), (0, 0)))
    nodes3 = nodes_p.reshape(NW, SCH, 16)
    neighs3 = neighs_p.reshape(NW, 4, NG // 4, DEG)

    self_raw, neigh_mean = _sc_gather_mean(
        embp, nodes3, neighs3, BP=BP, BPW=BPW, D=D, DEG=DEG, SCH=SCH, NG=NG)

    RB = 8
    for cand in range(512, 7, -8):
        if B % cand == 0:
            RB = cand
            break

    WaT = W1[:, :D].T
    WbT = W1[:, D:].T
    b2 = b1.reshape(1, D)
    return _tc_combine(self_raw, neigh_mean, WaT, WbT, b2, B=B, D=D, RB=RB)
